# baseline (speedup 1.0000x reference)
import functools

import jax
import jax.numpy as jnp
from jax import lax
from jax.experimental import pallas as pl
from jax.experimental.pallas import tpu as pltpu
from jax.experimental.pallas import tpu_sc as plsc

FIELD0 = 1000000
B = 4096
IO_DIM = 2048
EMBED_DIM = 64
TROWS = 2 * FIELD0
LGRP = TROWS // 128

_NC, _NS, _L = 2, 16, 16
_NW = _NC * _NS
_BPW = B // _NW

_BT = 512
_NT = B // _BT


def _sc_gather_body(xT_hbm, tflat_hbm, out_hbm,
                    uu_v, ui_v, addr_v, rows_v, sem):
    wid = lax.axis_index("s") * _NC + lax.axis_index("c")
    base = wid * _BPW
    pltpu.sync_copy(xT_hbm.at[0, pl.ds(base, _BPW)], uu_v)
    pltpu.sync_copy(xT_hbm.at[1, pl.ds(base, _BPW)], ui_v)
    for j in range(_BPW // _L):
        sl = pl.ds(j * _L, _L)
        ru = uu_v[sl]
        uu_v[sl] = (lax.shift_left(lax.shift_right_logical(ru, 7), 10)
                    + (ru & 127))
        ri = ui_v[sl] + FIELD0
        ui_v[sl] = (lax.shift_left(lax.shift_right_logical(ri, 7), 10)
                    + (ri & 127))

    def _emit(e, carry):
        te = (lax.shift_right_logical(e, 3) * (LGRP * 1024)
              + (e & 7) * 128)
        for j in range(_BPW // _L):
            u_sl = pl.ds(j * _L, _L)
            addr_v[pl.ds(e * _BPW + j * _L, _L)] = uu_v[u_sl] + te
            addr_v[pl.ds((e + EMBED_DIM) * _BPW + j * _L, _L)] = (
                ui_v[u_sl] + te)
        return carry

    lax.fori_loop(0, EMBED_DIM, _emit, 0)

    pltpu.async_copy(tflat_hbm.at[addr_v], rows_v, sem).wait()
    pltpu.sync_copy(rows_v, out_hbm.at[wid])


_sc_gather = functools.partial(
    pl.kernel,
    mesh=plsc.VectorSubcoreMesh(
        core_axis_name="c", subcore_axis_name="s",
        num_cores=_NC, num_subcores=_NS),
    out_type=jax.ShapeDtypeStruct((_NW, 2 * EMBED_DIM * _BPW), jnp.float32),
    scratch_types=[
        pltpu.VMEM((_BPW,), jnp.int32),
        pltpu.VMEM((_BPW,), jnp.int32),
        pltpu.VMEM((2 * EMBED_DIM * _BPW,), jnp.int32),
        pltpu.VMEM((2 * EMBED_DIM * _BPW,), jnp.float32),
        pltpu.SemaphoreType.DMA,
    ],
)(_sc_gather_body)


def _tc1_body(ux_ref, ix_ref, ueW_ref, ueb_ref, ieW_ref, ieb_ref,
              W1_ref, b1_ref, out_ref):
    uz = jnp.maximum(
        jnp.dot(ux_ref[...], ueW_ref[...], preferred_element_type=jnp.float32)
        + ueb_ref[...], 0.0)
    iz = jnp.maximum(
        jnp.dot(ix_ref[...], ieW_ref[...], preferred_element_type=jnp.float32)
        + ieb_ref[...], 0.0)
    out_ref[...] = (
        jnp.dot(uz, W1_ref[0:64, :], preferred_element_type=jnp.float32)
        + jnp.dot(iz, W1_ref[128:192, :], preferred_element_type=jnp.float32)
        + b1_ref[...])


def _tc1_call(user_x, item_x, ue_W, ue_b, ie_W, ie_b, W1, b1):
    full = lambda shape: pl.BlockSpec(shape, lambda t: (0, 0))
    return pl.pallas_call(
        _tc1_body,
        grid=(_NT,),
        in_specs=[
            pl.BlockSpec((_BT, IO_DIM), lambda t: (t, 0)),
            pl.BlockSpec((_BT, IO_DIM), lambda t: (t, 0)),
            full((IO_DIM, 64)), full((1, 64)),
            full((IO_DIM, 64)), full((1, 64)),
            full((256, 256)), full((1, 256)),
        ],
        out_specs=pl.BlockSpec((_BT, 256), lambda t: (t, 0)),
        out_shape=jax.ShapeDtypeStruct((B, 256), jnp.float32),
    )(user_x, item_x, ue_W, ue_b, ie_W, ie_b, W1, b1)


def _tc2_body(p1_ref, embT_ref, W1_ref, g1_ref, be1_ref,
              W2_ref, b2_ref, g2_ref, be2_ref, W3_ref, b3_ref, out_ref):
    cdims = (((0,), (0,)), ((), ()))
    embc = (lax.dot_general(embT_ref[0:64, :], W1_ref[64:128, :], cdims,
                            preferred_element_type=jnp.float32)
            + lax.dot_general(embT_ref[64:128, :], W1_ref[192:256, :], cdims,
                              preferred_element_type=jnp.float32))
    hp = p1_ref[...] + embc
    m1 = jnp.mean(hp, axis=0, keepdims=True)
    v1 = jnp.mean((hp - m1) ** 2, axis=0, keepdims=True)
    h1 = jnp.maximum(
        g1_ref[...] * (hp - m1) * lax.rsqrt(v1 + 1e-5) + be1_ref[...], 0.0)
    h2pre = (jnp.dot(h1, W2_ref[...], preferred_element_type=jnp.float32)
             + b2_ref[...])
    m2 = jnp.mean(h2pre, axis=0, keepdims=True)
    v2 = jnp.mean((h2pre - m2) ** 2, axis=0, keepdims=True)
    h2 = jnp.maximum(
        g2_ref[...] * (h2pre - m2) * lax.rsqrt(v2 + 1e-5) + be2_ref[...], 0.0)
    out_ref[...] = jnp.maximum(
        jnp.dot(h2, W3_ref[...], preferred_element_type=jnp.float32)
        + b3_ref[...], 0.0)


def _tc2_call(p1, embT, W1, g1, be1, W2, b2, g2, be2, W3, b3):
    return pl.pallas_call(
        _tc2_body,
        out_shape=jax.ShapeDtypeStruct((B, 1), jnp.float32),
    )(p1, embT, W1, g1, be1, W2, b2, g2, be2, W3, b3)


def kernel(x, user_x, item_x, emb_table, ue_W, ue_b, ie_W, ie_b,
           W1, b1, g1, be1, W2, b2, g2, be2, W3, b3):
    xT = x.astype(jnp.int32).T
    tflat = (emb_table.T.reshape(8, 8, LGRP, 128)
             .transpose(0, 2, 1, 3).reshape(TROWS * EMBED_DIM))
    raw = _sc_gather(xT, tflat)
    embT = (raw.reshape(_NW, 2 * EMBED_DIM, _BPW)
            .transpose(1, 0, 2).reshape(2 * EMBED_DIM, B))
    r2 = lambda a: a.reshape(1, -1)
    p1 = _tc1_call(user_x, item_x, ue_W, r2(ue_b), ie_W, r2(ie_b), W1, r2(b1))
    return _tc2_call(p1, embT, W1, r2(g1), r2(be1),
                     W2, r2(b2), r2(g2), r2(be2), W3, r2(b3))


# speedup vs baseline: 1.1372x; 1.1372x over previous
import functools

import jax
import jax.numpy as jnp
from jax import lax
from jax.experimental import pallas as pl
from jax.experimental.pallas import tpu as pltpu
from jax.experimental.pallas import tpu_sc as plsc

FIELD0 = 1000000
B = 4096
IO_DIM = 2048
EMBED_DIM = 64
TROWS = 2 * FIELD0
LGRP = TROWS // 128

_NC, _NS, _L = 2, 16, 16
_NW = _NC * _NS
_BPW = B // _NW

_BT = 512
_NT = B // _BT


def _sc_gather_body(xT_hbm, tflat_hbm, out_hbm,
                    uu_v, ui_v, addr_v, rows_v, sem):
    wid = lax.axis_index("s") * _NC + lax.axis_index("c")
    base = wid * _BPW
    pltpu.sync_copy(xT_hbm.at[0, pl.ds(base, _BPW)], uu_v)
    pltpu.sync_copy(xT_hbm.at[1, pl.ds(base, _BPW)], ui_v)
    for j in range(_BPW // _L):
        sl = pl.ds(j * _L, _L)
        ru = uu_v[sl]
        uu_v[sl] = (lax.shift_left(lax.shift_right_logical(ru, 7), 10)
                    + (ru & 127))
        ri = ui_v[sl] + FIELD0
        ui_v[sl] = (lax.shift_left(lax.shift_right_logical(ri, 7), 10)
                    + (ri & 127))

    def _emit(e, carry):
        te = (lax.shift_right_logical(e, 3) * (LGRP * 1024)
              + (e & 7) * 128)
        for j in range(_BPW // _L):
            u_sl = pl.ds(j * _L, _L)
            addr_v[pl.ds(e * _BPW + j * _L, _L)] = uu_v[u_sl] + te
            addr_v[pl.ds((e + EMBED_DIM) * _BPW + j * _L, _L)] = (
                ui_v[u_sl] + te)
        return carry

    lax.fori_loop(0, EMBED_DIM, _emit, 0)

    copies = []
    for d in range(2 * EMBED_DIM):
        copies.append(pltpu.async_copy(
            tflat_hbm.at[addr_v.at[pl.ds(d * _BPW, _BPW)]],
            rows_v.at[d], sem))
    for c in copies:
        c.wait()
    pltpu.sync_copy(rows_v, out_hbm.at[:, pl.ds(base, _BPW)])


_sc_gather = functools.partial(
    pl.kernel,
    mesh=plsc.VectorSubcoreMesh(
        core_axis_name="c", subcore_axis_name="s",
        num_cores=_NC, num_subcores=_NS),
    out_type=jax.ShapeDtypeStruct((2 * EMBED_DIM, B), jnp.float32),
    scratch_types=[
        pltpu.VMEM((_BPW,), jnp.int32),
        pltpu.VMEM((_BPW,), jnp.int32),
        pltpu.VMEM((2 * EMBED_DIM * _BPW,), jnp.int32),
        pltpu.VMEM((2 * EMBED_DIM, _BPW), jnp.float32),
        pltpu.SemaphoreType.DMA,
    ],
)(_sc_gather_body)


def _tc1_body(ux_ref, ix_ref, ueWT_ref, ueb_ref, ieWT_ref, ieb_ref,
              W1_ref, b1_ref, out_ref):
    cT = (((1,), (1,)), ((), ()))
    uz = jnp.maximum(
        lax.dot_general(ux_ref[...], ueWT_ref[...], cT,
                        preferred_element_type=jnp.float32)
        + ueb_ref[...], 0.0)
    iz = jnp.maximum(
        lax.dot_general(ix_ref[...], ieWT_ref[...], cT,
                        preferred_element_type=jnp.float32)
        + ieb_ref[...], 0.0)
    out_ref[...] = (
        jnp.dot(uz, W1_ref[0:64, :], preferred_element_type=jnp.float32)
        + jnp.dot(iz, W1_ref[128:192, :], preferred_element_type=jnp.float32)
        + b1_ref[...])


def _tc1_call(user_x, item_x, ue_WT, ue_b, ie_WT, ie_b, W1, b1):
    full = lambda shape: pl.BlockSpec(shape, lambda t: (0, 0))
    return pl.pallas_call(
        _tc1_body,
        grid=(_NT,),
        in_specs=[
            pl.BlockSpec((_BT, IO_DIM), lambda t: (t, 0)),
            pl.BlockSpec((_BT, IO_DIM), lambda t: (t, 0)),
            full((64, IO_DIM)), full((1, 64)),
            full((64, IO_DIM)), full((1, 64)),
            full((256, 256)), full((1, 256)),
        ],
        out_specs=pl.BlockSpec((_BT, 256), lambda t: (t, 0)),
        out_shape=jax.ShapeDtypeStruct((B, 256), jnp.float32),
    )(user_x, item_x, ue_WT, ue_b, ie_WT, ie_b, W1, b1)


def _tc2_body(p1_ref, embT_ref, W1_ref, g1_ref, be1_ref,
              W2_ref, b2_ref, g2_ref, be2_ref, W3_ref, b3_ref, out_ref):
    cdims = (((0,), (0,)), ((), ()))
    embc = (lax.dot_general(embT_ref[0:64, :], W1_ref[64:128, :], cdims,
                            preferred_element_type=jnp.float32)
            + lax.dot_general(embT_ref[64:128, :], W1_ref[192:256, :], cdims,
                              preferred_element_type=jnp.float32))
    hp = p1_ref[...] + embc
    m1 = jnp.mean(hp, axis=0, keepdims=True)
    v1 = jnp.mean((hp - m1) ** 2, axis=0, keepdims=True)
    h1 = jnp.maximum(
        g1_ref[...] * (hp - m1) * lax.rsqrt(v1 + 1e-5) + be1_ref[...], 0.0)
    h2pre = (jnp.dot(h1, W2_ref[...], preferred_element_type=jnp.float32)
             + b2_ref[...])
    m2 = jnp.mean(h2pre, axis=0, keepdims=True)
    v2 = jnp.mean((h2pre - m2) ** 2, axis=0, keepdims=True)
    h2 = jnp.maximum(
        g2_ref[...] * (h2pre - m2) * lax.rsqrt(v2 + 1e-5) + be2_ref[...], 0.0)
    out_ref[...] = jnp.maximum(
        lax.dot_general(W3_ref[...], h2, (((0,), (1,)), ((), ())),
                        preferred_element_type=jnp.float32).reshape(1, B)
        + b3_ref[...], 0.0)


def _tc2_call(p1, embT, W1, g1, be1, W2, b2, g2, be2, W3, b3):
    return pl.pallas_call(
        _tc2_body,
        out_shape=jax.ShapeDtypeStruct((1, B), jnp.float32),
    )(p1, embT, W1, g1, be1, W2, b2, g2, be2, W3, b3)


def kernel(x, user_x, item_x, emb_table, ue_W, ue_b, ie_W, ie_b,
           W1, b1, g1, be1, W2, b2, g2, be2, W3, b3):
    xT = x.astype(jnp.int32).T
    tflat = (emb_table.T.reshape(8, 8, LGRP, 128)
             .transpose(0, 2, 1, 3).reshape(TROWS * EMBED_DIM))
    embT = _sc_gather(xT, tflat)
    r2 = lambda a: a.reshape(1, -1)
    p1 = _tc1_call(user_x, item_x, ue_W.T, r2(ue_b), ie_W.T, r2(ie_b),
                   W1, r2(b1))
    yT = _tc2_call(p1, embT, W1, r2(g1), r2(be1),
                   W2, r2(b2), r2(g2), r2(be2), W3, r2(b3))
    return yT.reshape(B, 1)


# speedup vs baseline: 1.1405x; 1.0029x over previous
import functools

import jax
import jax.numpy as jnp
from jax import lax
from jax.experimental import pallas as pl
from jax.experimental.pallas import tpu as pltpu
from jax.experimental.pallas import tpu_sc as plsc

FIELD0 = 1000000
B = 4096
IO_DIM = 2048
EMBED_DIM = 64
TROWS = 2 * FIELD0
LGRP = TROWS // 128

_NC, _NS, _L = 2, 16, 16
_NW = _NC * _NS
_BPW = B // _NW

_BT = 512
_NT = B // _BT


def _sc_gather_body(xT_hbm, tflat_hbm, out_hbm,
                    uu_v, ui_v, addr_v, rows_v, sem):
    wid = lax.axis_index("s") * _NC + lax.axis_index("c")
    base = wid * _BPW
    pltpu.sync_copy(xT_hbm.at[0, pl.ds(base, _BPW)], uu_v)
    pltpu.sync_copy(xT_hbm.at[1, pl.ds(base, _BPW)], ui_v)
    for j in range(_BPW // _L):
        sl = pl.ds(j * _L, _L)
        ru = uu_v[sl]
        uu_v[sl] = (lax.shift_left(lax.shift_right_logical(ru, 7), 10)
                    + (ru & 127))
        ri = ui_v[sl] + FIELD0
        ui_v[sl] = (lax.shift_left(lax.shift_right_logical(ri, 7), 10)
                    + (ri & 127))

    def _emit(e, carry):
        te = (lax.shift_right_logical(e, 3) * (LGRP * 1024)
              + (e & 7) * 128)
        for j in range(_BPW // _L):
            u_sl = pl.ds(j * _L, _L)
            addr_v[pl.ds(e * _BPW + j * _L, _L)] = uu_v[u_sl] + te
            addr_v[pl.ds((e + EMBED_DIM) * _BPW + j * _L, _L)] = (
                ui_v[u_sl] + te)
        return carry

    lax.fori_loop(0, EMBED_DIM, _emit, 0)

    copies = []
    for d in range(2 * EMBED_DIM):
        copies.append(pltpu.async_copy(
            tflat_hbm.at[addr_v.at[pl.ds(d * _BPW, _BPW)]],
            rows_v.at[d], sem))
    for c in copies:
        c.wait()
    pltpu.sync_copy(rows_v, out_hbm.at[:, pl.ds(base, _BPW)])


_sc_gather = functools.partial(
    pl.kernel,
    mesh=plsc.VectorSubcoreMesh(
        core_axis_name="c", subcore_axis_name="s",
        num_cores=_NC, num_subcores=_NS),
    out_type=jax.ShapeDtypeStruct((2 * EMBED_DIM, B), jnp.float32),
    scratch_types=[
        pltpu.VMEM((_BPW,), jnp.int32),
        pltpu.VMEM((_BPW,), jnp.int32),
        pltpu.VMEM((2 * EMBED_DIM * _BPW,), jnp.int32),
        pltpu.VMEM((2 * EMBED_DIM, _BPW), jnp.float32),
        pltpu.SemaphoreType.DMA,
    ],
)(_sc_gather_body)


def _tc1_body(ux_ref, ix_ref, ueWT_ref, ueb_ref, ieWT_ref, ieb_ref,
              W1_ref, b1_ref, out_ref):
    cT = (((1,), (1,)), ((), ()))
    uz = jnp.maximum(
        lax.dot_general(ux_ref[...], ueWT_ref[...], cT,
                        preferred_element_type=jnp.float32)
        + ueb_ref[...], 0.0)
    iz = jnp.maximum(
        lax.dot_general(ix_ref[...], ieWT_ref[...], cT,
                        preferred_element_type=jnp.float32)
        + ieb_ref[...], 0.0)
    out_ref[...] = (
        jnp.dot(uz, W1_ref[0:64, :], preferred_element_type=jnp.float32)
        + jnp.dot(iz, W1_ref[128:192, :], preferred_element_type=jnp.float32)
        + b1_ref[...])


def _tc1_call(user_x, item_x, ue_WT, ue_b, ie_WT, ie_b, W1, b1):
    full = lambda shape: pl.BlockSpec(shape, lambda t: (0, 0))
    return pl.pallas_call(
        _tc1_body,
        grid=(_NT,),
        in_specs=[
            pl.BlockSpec((_BT, IO_DIM), lambda t: (t, 0)),
            pl.BlockSpec((_BT, IO_DIM), lambda t: (t, 0)),
            full((64, IO_DIM)), full((1, 64)),
            full((64, IO_DIM)), full((1, 64)),
            full((256, 256)), full((1, 256)),
        ],
        out_specs=pl.BlockSpec((_BT, 256), lambda t: (t, 0)),
        out_shape=jax.ShapeDtypeStruct((B, 256), jnp.float32),
    )(user_x, item_x, ue_WT, ue_b, ie_WT, ie_b, W1, b1)


def _tc2_body(p1_ref, embT_ref, W1_ref, g1_ref, be1_ref,
              W2_ref, b2_ref, g2_ref, be2_ref, W3_ref, b3_ref, out_ref):
    cdims = (((0,), (0,)), ((), ()))
    bf = jnp.bfloat16
    embc = (lax.dot_general(embT_ref[0:64, :].astype(bf),
                            W1_ref[64:128, :].astype(bf), cdims,
                            preferred_element_type=jnp.float32)
            + lax.dot_general(embT_ref[64:128, :].astype(bf),
                              W1_ref[192:256, :].astype(bf), cdims,
                              preferred_element_type=jnp.float32))
    hp = p1_ref[...] + embc
    m1 = jnp.mean(hp, axis=0, keepdims=True)
    v1 = jnp.mean((hp - m1) ** 2, axis=0, keepdims=True)
    h1 = jnp.maximum(
        g1_ref[...] * (hp - m1) * lax.rsqrt(v1 + 1e-5) + be1_ref[...], 0.0)
    h2pre = (jnp.dot(h1.astype(bf), W2_ref[...].astype(bf),
                     preferred_element_type=jnp.float32)
             + b2_ref[...])
    m2 = jnp.mean(h2pre, axis=0, keepdims=True)
    v2 = jnp.mean((h2pre - m2) ** 2, axis=0, keepdims=True)
    h2 = jnp.maximum(
        g2_ref[...] * (h2pre - m2) * lax.rsqrt(v2 + 1e-5) + be2_ref[...], 0.0)
    out_ref[...] = jnp.maximum(
        lax.dot_general(W3_ref[...], h2, (((0,), (1,)), ((), ())),
                        preferred_element_type=jnp.float32).reshape(1, B)
        + b3_ref[...], 0.0)


def _tc2_call(p1, embT, W1, g1, be1, W2, b2, g2, be2, W3, b3):
    return pl.pallas_call(
        _tc2_body,
        out_shape=jax.ShapeDtypeStruct((1, B), jnp.float32),
    )(p1, embT, W1, g1, be1, W2, b2, g2, be2, W3, b3)


def kernel(x, user_x, item_x, emb_table, ue_W, ue_b, ie_W, ie_b,
           W1, b1, g1, be1, W2, b2, g2, be2, W3, b3):
    xT = x.astype(jnp.int32).T
    tflat = (emb_table.T.reshape(8, 8, LGRP, 128)
             .transpose(0, 2, 1, 3).reshape(TROWS * EMBED_DIM))
    embT = _sc_gather(xT, tflat)
    r2 = lambda a: a.reshape(1, -1)
    p1 = _tc1_call(user_x, item_x, ue_W.T, r2(ue_b), ie_W.T, r2(ie_b),
                   W1, r2(b1))
    yT = _tc2_call(p1, embT, W1, r2(g1), r2(be1),
                   W2, r2(b2), r2(g2), r2(be2), W3, r2(b3))
    return yT.reshape(B, 1)
